# Initial kernel scaffold; baseline (speedup 1.0000x reference)
import jax
import jax.numpy as jnp
from jax.experimental import pallas as pl


def kernel(patch_logits):
    raise NotImplementedError("write your pallas kernel here")


# speedup vs baseline: 8.9829x; 8.9829x over previous
import functools

import jax
import jax.numpy as jnp
from jax import lax
from jax.experimental import pallas as pl

_K_RATIO = 0.1
_MIN_K = 5


def _topk_mean_body(x_ref, o_ref, *, k):
    x = x_ref[...]
    kf = jnp.float32(k)
    bu = lax.bitcast_convert_type(x, jnp.uint32)
    sign = bu >> jnp.uint32(31)
    mask = (sign * jnp.uint32(0xFFFFFFFF)) | jnp.uint32(0x80000000)
    key = bu ^ mask

    def step(i, prefix):
        bit = jnp.uint32(31) - i.astype(jnp.uint32)
        cand = prefix | (jnp.uint32(1) << bit)
        cnt = jnp.sum((key >= cand).astype(jnp.float32), axis=1, keepdims=True)
        return jnp.where(cnt >= kf, cand, prefix)

    rows = x.shape[0]
    t = lax.fori_loop(0, 32, step, jnp.zeros((rows, 1), jnp.uint32))

    gt = key > t
    cnt_gt = jnp.sum(gt.astype(jnp.float32), axis=1, keepdims=True)
    sum_gt = jnp.sum(jnp.where(gt, x, 0.0), axis=1, keepdims=True)
    tb = jnp.where((t & jnp.uint32(0x80000000)) != 0,
                   t ^ jnp.uint32(0x80000000), ~t)
    tval = lax.bitcast_convert_type(tb, jnp.float32)
    o_ref[...] = (sum_gt + (kf - cnt_gt) * tval) / kf


def kernel(patch_logits):
    if patch_logits.ndim == 4:
        b = patch_logits.shape[0]
        patch_logits = patch_logits.reshape(b, -1)
    rows, n = patch_logits.shape
    k = max(_MIN_K, int(-(-n * _K_RATIO // 1)))
    block_rows = 8 if rows % 8 == 0 else rows
    grid = rows // block_rows
    return pl.pallas_call(
        functools.partial(_topk_mean_body, k=k),
        grid=(grid,),
        in_specs=[pl.BlockSpec((block_rows, n), lambda i: (i, 0))],
        out_specs=pl.BlockSpec((block_rows, 1), lambda i: (i, 0)),
        out_shape=jax.ShapeDtypeStruct((rows, 1), jnp.float32),
    )(patch_logits)
